# baseline (speedup 1.0000x reference)
import jax
import jax.numpy as jnp
from jax import lax
from jax.experimental import pallas as pl
from jax.experimental.pallas import tpu as pltpu
from jax.experimental.pallas import tpu_sc as plsc

TOPK_K = 256
ROWS = 128
N = 32768
CHUNKS = N // 16
MASK31 = 0x7FFFFFFF
BIG = 0x7FFFFFFF
UNROLL = 8
HCOPIES = 4

_H1_OFF = 0
_H2_OFF = 2048
_H3_OFF = 3072
_HTOT = 4096


def _find_crossing(hists, off, nbuckets, m):
    lanes = lax.broadcasted_iota(jnp.int32, (16,), 0)
    zeros16 = jnp.zeros((16,), jnp.int32)

    def body(c, carry):
        found, b, pbef, acc = carry
        vs = [h[pl.ds(off + c * 16, 16)] for h in hists]
        v = vs[0]
        for u in range(1, len(hists)):
            v = v + vs[u]
        for h in hists:
            h[pl.ds(off + c * 16, 16)] = zeros16
        cs = plsc.cumsum(v) + acc
        pb = cs - v
        cross = cs > m
        lane = jnp.min(jnp.where(cross, lanes, BIG))
        this_found = (lane < 16).astype(jnp.int32)
        use = this_found * (1 - found)
        b = jnp.where(use == 1, c * 16 + lane, b)
        pbef = jnp.where(use == 1, jnp.min(jnp.where(cross, pb, BIG)), pbef)
        found = jnp.maximum(found, this_found)
        acc = acc + jnp.sum(v)
        return found, b, pbef, acc

    init = (jnp.int32(0), jnp.int32(0), jnp.int32(0), jnp.int32(0))
    _, b, pbef, _ = lax.fori_loop(0, nbuckets // 16, body, init)
    return b, pbef


def _make_sc_kernel():
    mesh = plsc.VectorSubcoreMesh(core_axis_name="c", subcore_axis_name="s")

    @lambda body: pl.kernel(
        body,
        out_type=jax.ShapeDtypeStruct((ROWS, N), jnp.float32),
        mesh=mesh,
        scratch_types=[
            pltpu.VMEM((N,), jnp.float32),
            pltpu.VMEM((N,), jnp.float32),
            pltpu.VMEM((N + 16,), jnp.int32),
        ]
        + [pltpu.VMEM((_HTOT,), jnp.int32) for _ in range(HCOPIES)],
        compiler_params=pltpu.CompilerParams(needs_layout_passes=False),
    )
    def sc_kernel(x_hbm, out_hbm, row_v, out_v, cand, *hists):
        wid = lax.axis_index("s") * 2 + lax.axis_index("c")
        ones = jnp.ones((16,), jnp.int32)
        zeros16 = jnp.zeros((16,), jnp.int32)
        lanes = lax.broadcasted_iota(jnp.int32, (16,), 0)

        def clr(c, _):
            for h in hists:
                h[pl.ds(c * 16, 16)] = zeros16
            return 0

        lax.fori_loop(0, _HTOT // 16, clr, 0)

        def per_row(j, _):
            row = wid * 4 + j
            pltpu.sync_copy(x_hbm.at[row], row_v)

            def s1(c, _):
                for u in range(UNROLL):
                    v = row_v[pl.ds(c * (16 * UNROLL) + u * 16, 16)]
                    bits = plsc.bitcast(v, jnp.int32) & MASK31
                    b = lax.shift_right_logical(bits, 20)
                    plsc.addupdate_scatter(hists[u % HCOPIES], [b], ones)
                return 0

            lax.fori_loop(0, CHUNKS // UNROLL, s1, 0)

            m1 = jnp.int32(N - TOPK_K)
            b1, pbef1 = _find_crossing(hists, _H1_OFF, 2048, m1)
            m2 = m1 - pbef1

            def s2(c, cnt):
                for u in range(UNROLL):
                    v = row_v[pl.ds(c * (16 * UNROLL) + u * 16, 16)]
                    bits = plsc.bitcast(v, jnp.int32) & MASK31
                    match = lax.shift_right_logical(bits, 20) == b1
                    plsc.store_compressed(cand.at[pl.ds(cnt, 16)], bits, mask=match)
                    cnt = cnt + jnp.sum(match.astype(jnp.int32))
                return cnt

            cnt = lax.fori_loop(0, CHUNKS // UNROLL, s2, jnp.int32(0))
            nc = (cnt + 15) // 16

            def h2(c, _):
                bits = cand[pl.ds(c * 16, 16)]
                valid = (c * 16 + lanes) < cnt
                b = (lax.shift_right_logical(bits, 10) & 1023) + _H2_OFF
                plsc.addupdate_scatter(hists[0], [b], ones, mask=valid)
                return 0

            lax.fori_loop(0, nc, h2, 0)
            b2, pbef2 = _find_crossing(hists, _H2_OFF, 1024, m2)
            m3 = m2 - pbef2
            pfx21 = (b1 << 10) | b2

            def s3(c, _):
                bits = cand[pl.ds(c * 16, 16)]
                valid = (c * 16 + lanes) < cnt
                match = valid & (lax.shift_right_logical(bits, 10) == pfx21)
                b = (bits & 1023) + _H3_OFF
                plsc.addupdate_scatter(hists[0], [b], ones, mask=match)
                return 0

            lax.fori_loop(0, nc, s3, 0)
            b3, _ = _find_crossing(hists, _H3_OFF, 1024, m3)
            thr = (pfx21 << 10) | b3

            def mk(c, _):
                for u in range(UNROLL):
                    v = row_v[pl.ds(c * (16 * UNROLL) + u * 16, 16)]
                    bits = plsc.bitcast(v, jnp.int32) & MASK31
                    out_v[pl.ds(c * (16 * UNROLL) + u * 16, 16)] = jnp.where(
                        bits >= thr, v, 0.0
                    )
                return 0

            lax.fori_loop(0, CHUNKS // UNROLL, mk, 0)
            pltpu.sync_copy(out_v, out_hbm.at[row])
            return 0

        lax.fori_loop(0, ROWS // 32, per_row, 0)

    return sc_kernel


_SC_KERNEL = _make_sc_kernel()


def kernel(x):
    return _SC_KERNEL(x)


# speedup vs baseline: 1.1092x; 1.1092x over previous
import jax
import jax.numpy as jnp
from jax import lax
from jax.experimental import pallas as pl
from jax.experimental.pallas import tpu as pltpu
from jax.experimental.pallas import tpu_sc as plsc

TOPK_K = 256
ROWS = 128
N = 32768
CHUNKS = N // 16
MASK31 = 0x7FFFFFFF
BIG = 0x7FFFFFFF
UNROLL = 8
HCOPIES = 4

_H1_OFF = 0
_H2_OFF = 2048
_H3_OFF = 3072
_HTOT = 4096


def _find_crossing(hists, off, nbuckets, m):
    lanes = lax.broadcasted_iota(jnp.int32, (16,), 0)
    zeros16 = jnp.zeros((16,), jnp.int32)

    def body(c, carry):
        found, b, pbef, acc = carry
        vs = [h[pl.ds(off + c * 16, 16)] for h in hists]
        v = vs[0]
        for u in range(1, len(hists)):
            v = v + vs[u]
        for h in hists:
            h[pl.ds(off + c * 16, 16)] = zeros16
        cs = plsc.cumsum(v) + acc
        pb = cs - v
        cross = cs > m
        lane = jnp.min(jnp.where(cross, lanes, BIG))
        this_found = (lane < 16).astype(jnp.int32)
        use = this_found * (1 - found)
        b = jnp.where(use == 1, c * 16 + lane, b)
        pbef = jnp.where(use == 1, jnp.min(jnp.where(cross, pb, BIG)), pbef)
        found = jnp.maximum(found, this_found)
        acc = acc + jnp.sum(v)
        return found, b, pbef, acc

    init = (jnp.int32(0), jnp.int32(0), jnp.int32(0), jnp.int32(0))
    _, b, pbef, _ = lax.fori_loop(0, nbuckets // 16, body, init)
    return b, pbef


def _make_sc_kernel():
    mesh = plsc.VectorSubcoreMesh(core_axis_name="c", subcore_axis_name="s")

    @lambda body: pl.kernel(
        body,
        out_type=jax.ShapeDtypeStruct((ROWS, N), jnp.float32),
        mesh=mesh,
        scratch_types=[
            pltpu.VMEM((N,), jnp.float32),
            pltpu.VMEM((N,), jnp.float32),
            pltpu.VMEM((N + 16,), jnp.int32),
        ]
        + [pltpu.VMEM((_HTOT,), jnp.int32) for _ in range(HCOPIES)],
        compiler_params=pltpu.CompilerParams(needs_layout_passes=False),
    )
    def sc_kernel(x_hbm, out_hbm, row_v, out_v, cand, *hists):
        wid = lax.axis_index("s") * 2 + lax.axis_index("c")
        ones = jnp.ones((16,), jnp.int32)
        zeros16 = jnp.zeros((16,), jnp.int32)
        lanes = lax.broadcasted_iota(jnp.int32, (16,), 0)

        def clr(c, _):
            for h in hists:
                h[pl.ds(c * 16, 16)] = zeros16
            return 0

        lax.fori_loop(0, _HTOT // 16, clr, 0)

        def per_row(j, _):
            row = wid * 4 + j
            pltpu.sync_copy(x_hbm.at[row], row_v)

            def s1(c, _):
                for u in range(UNROLL):
                    v = row_v[pl.ds(c * (16 * UNROLL) + u * 16, 16)]
                    bits = plsc.bitcast(v, jnp.int32) & MASK31
                    b = lax.shift_right_logical(bits, 20)
                    plsc.addupdate_scatter(hists[u % HCOPIES], [b], ones)
                return 0

            lax.fori_loop(0, CHUNKS // UNROLL, s1, 0)

            m1 = jnp.int32(N - TOPK_K)
            b1, pbef1 = _find_crossing(hists, _H1_OFF, 2048, m1)
            m2 = m1 - pbef1

            def s2(c, cnt):
                for u in range(UNROLL):
                    v = row_v[pl.ds(c * (16 * UNROLL) + u * 16, 16)]
                    bits = plsc.bitcast(v, jnp.int32) & MASK31
                    match = lax.shift_right_logical(bits, 20) == b1
                    b = (lax.shift_right_logical(bits, 10) & 1023) + _H2_OFF
                    plsc.addupdate_scatter(hists[u % HCOPIES], [b], ones, mask=match)
                    plsc.store_compressed(cand.at[pl.ds(cnt, 16)], bits, mask=match)
                    cnt = cnt + jnp.sum(match.astype(jnp.int32))
                return cnt

            cnt = lax.fori_loop(0, CHUNKS // UNROLL, s2, jnp.int32(0))
            nc = (cnt + 15) // 16
            b2, pbef2 = _find_crossing(hists, _H2_OFF, 1024, m2)
            m3 = m2 - pbef2
            pfx21 = (b1 << 10) | b2

            def s3(c, _):
                bits = cand[pl.ds(c * 16, 16)]
                valid = (c * 16 + lanes) < cnt
                match = valid & (lax.shift_right_logical(bits, 10) == pfx21)
                b = (bits & 1023) + _H3_OFF
                plsc.addupdate_scatter(hists[0], [b], ones, mask=match)
                return 0

            lax.fori_loop(0, nc, s3, 0)
            b3, _ = _find_crossing(hists, _H3_OFF, 1024, m3)
            thr = (pfx21 << 10) | b3

            def mk(c, _):
                for u in range(UNROLL):
                    v = row_v[pl.ds(c * (16 * UNROLL) + u * 16, 16)]
                    bits = plsc.bitcast(v, jnp.int32) & MASK31
                    out_v[pl.ds(c * (16 * UNROLL) + u * 16, 16)] = jnp.where(
                        bits >= thr, v, 0.0
                    )
                return 0

            lax.fori_loop(0, CHUNKS // UNROLL, mk, 0)
            pltpu.sync_copy(out_v, out_hbm.at[row])
            return 0

        lax.fori_loop(0, ROWS // 32, per_row, 0)

    return sc_kernel


_SC_KERNEL = _make_sc_kernel()


def kernel(x):
    return _SC_KERNEL(x)


# speedup vs baseline: 1.6559x; 1.4930x over previous
import jax
import jax.numpy as jnp
from jax import lax
from jax.experimental import pallas as pl
from jax.experimental.pallas import tpu as pltpu
from jax.experimental.pallas import tpu_sc as plsc

TOPK_K = 256
ROWS = 128
N = 32768
CHUNKS = N // 16
MASK31 = 0x7FFFFFFF
BIG = 0x7FFFFFFF
UNROLL = 8
HCOPIES = 4
SUBCORES = 32
SC_ROWS = 64

_H1_OFF = 0
_H2_OFF = 2048
_H3_OFF = 3072
_HTOT = 4096


def _find_crossing(hists, off, nbuckets, m):
    lanes = lax.broadcasted_iota(jnp.int32, (16,), 0)
    zeros16 = jnp.zeros((16,), jnp.int32)

    def body(c, carry):
        found, b, pbef, acc = carry
        vs = [h[pl.ds(off + c * 16, 16)] for h in hists]
        v = vs[0]
        for u in range(1, len(hists)):
            v = v + vs[u]
        for h in hists:
            h[pl.ds(off + c * 16, 16)] = zeros16
        cs = plsc.cumsum(v) + acc
        pb = cs - v
        cross = cs > m
        lane = jnp.min(jnp.where(cross, lanes, BIG))
        this_found = (lane < 16).astype(jnp.int32)
        use = this_found * (1 - found)
        b = jnp.where(use == 1, c * 16 + lane, b)
        pbef = jnp.where(use == 1, jnp.min(jnp.where(cross, pb, BIG)), pbef)
        found = jnp.maximum(found, this_found)
        acc = acc + jnp.sum(v)
        return found, b, pbef, acc

    init = (jnp.int32(0), jnp.int32(0), jnp.int32(0), jnp.int32(0))
    _, b, pbef, _ = lax.fori_loop(0, nbuckets // 16, body, init)
    return b, pbef


def _make_sc_kernel(sc_rows):
    rows_per = sc_rows // SUBCORES
    mesh = plsc.VectorSubcoreMesh(core_axis_name="c", subcore_axis_name="s")

    @lambda body: pl.kernel(
        body,
        out_type=jax.ShapeDtypeStruct((sc_rows, N), jnp.float32),
        mesh=mesh,
        scratch_types=[
            pltpu.VMEM((N,), jnp.float32),
            pltpu.VMEM((N,), jnp.float32),
            pltpu.VMEM((N + 16,), jnp.int32),
        ]
        + [pltpu.VMEM((_HTOT,), jnp.int32) for _ in range(HCOPIES)],
        compiler_params=pltpu.CompilerParams(needs_layout_passes=False),
    )
    def sc_kernel(x_hbm, out_hbm, row_v, out_v, cand, *hists):
        wid = lax.axis_index("s") * 2 + lax.axis_index("c")
        ones = jnp.ones((16,), jnp.int32)
        zeros16 = jnp.zeros((16,), jnp.int32)
        lanes = lax.broadcasted_iota(jnp.int32, (16,), 0)

        def clr(c, _):
            for h in hists:
                h[pl.ds(c * 16, 16)] = zeros16
            return 0

        lax.fori_loop(0, _HTOT // 16, clr, 0)

        def per_row(j, _):
            row = wid * rows_per + j
            pltpu.sync_copy(x_hbm.at[row], row_v)

            def s1(c, _):
                for u in range(UNROLL):
                    v = row_v[pl.ds(c * (16 * UNROLL) + u * 16, 16)]
                    bits = plsc.bitcast(v, jnp.int32) & MASK31
                    b = lax.shift_right_logical(bits, 20)
                    plsc.addupdate_scatter(hists[u % HCOPIES], [b], ones)
                return 0

            lax.fori_loop(0, CHUNKS // UNROLL, s1, 0)

            m1 = jnp.int32(N - TOPK_K)
            b1, pbef1 = _find_crossing(hists, _H1_OFF, 2048, m1)
            m2 = m1 - pbef1

            def s2(c, cnt):
                for u in range(UNROLL):
                    v = row_v[pl.ds(c * (16 * UNROLL) + u * 16, 16)]
                    bits = plsc.bitcast(v, jnp.int32) & MASK31
                    match = lax.shift_right_logical(bits, 20) == b1
                    b = (lax.shift_right_logical(bits, 10) & 1023) + _H2_OFF
                    plsc.addupdate_scatter(hists[u % HCOPIES], [b], ones, mask=match)
                    plsc.store_compressed(cand.at[pl.ds(cnt, 16)], bits, mask=match)
                    cnt = cnt + jnp.sum(match.astype(jnp.int32))
                return cnt

            cnt = lax.fori_loop(0, CHUNKS // UNROLL, s2, jnp.int32(0))
            nc = (cnt + 15) // 16
            b2, pbef2 = _find_crossing(hists, _H2_OFF, 1024, m2)
            m3 = m2 - pbef2
            pfx21 = (b1 << 10) | b2

            def s3(c, _):
                bits = cand[pl.ds(c * 16, 16)]
                valid = (c * 16 + lanes) < cnt
                match = valid & (lax.shift_right_logical(bits, 10) == pfx21)
                b = (bits & 1023) + _H3_OFF
                plsc.addupdate_scatter(hists[0], [b], ones, mask=match)
                return 0

            lax.fori_loop(0, nc, s3, 0)
            b3, _ = _find_crossing(hists, _H3_OFF, 1024, m3)
            thr = (pfx21 << 10) | b3

            def mk(c, _):
                for u in range(UNROLL):
                    v = row_v[pl.ds(c * (16 * UNROLL) + u * 16, 16)]
                    bits = plsc.bitcast(v, jnp.int32) & MASK31
                    out_v[pl.ds(c * (16 * UNROLL) + u * 16, 16)] = jnp.where(
                        bits >= thr, v, 0.0
                    )
                return 0

            lax.fori_loop(0, CHUNKS // UNROLL, mk, 0)
            pltpu.sync_copy(out_v, out_hbm.at[row])
            return 0

        lax.fori_loop(0, rows_per, per_row, 0)

    return sc_kernel


_SC_KERNEL = _make_sc_kernel(SC_ROWS)


def _tc_body(x_ref, o_ref):
    xv = x_ref[...]
    bits = lax.bitcast_convert_type(xv, jnp.int32) & jnp.int32(MASK31)
    br = xv.shape[0]

    def step(i, t):
        candbit = t | lax.shift_left(jnp.int32(1), 30 - i)
        cnt = jnp.sum((bits >= candbit).astype(jnp.int32), axis=1, keepdims=True)
        return jnp.where(cnt >= TOPK_K, candbit, t)

    t = lax.fori_loop(0, 31, step, jnp.zeros((br, 1), jnp.int32))
    o_ref[...] = jnp.where(bits >= t, xv, 0.0)


def _tc_call(x):
    rows, n = x.shape
    br = 8
    return pl.pallas_call(
        _tc_body,
        grid=(rows // br,),
        in_specs=[pl.BlockSpec((br, n), lambda i: (i, 0))],
        out_specs=pl.BlockSpec((br, n), lambda i: (i, 0)),
        out_shape=jax.ShapeDtypeStruct(x.shape, x.dtype),
    )(x)


def kernel(x):
    out_sc = _SC_KERNEL(x[ROWS - SC_ROWS:])
    out_tc = _tc_call(x[: ROWS - SC_ROWS])
    return jnp.concatenate([out_tc, out_sc], axis=0)


# speedup vs baseline: 1.6789x; 1.0139x over previous
import jax
import jax.numpy as jnp
from jax import lax
from jax.experimental import pallas as pl
from jax.experimental.pallas import tpu as pltpu
from jax.experimental.pallas import tpu_sc as plsc

TOPK_K = 256
ROWS = 128
N = 32768
CHUNKS = N // 16
MASK31 = 0x7FFFFFFF
BIG = 0x7FFFFFFF
UNROLL = 8
HCOPIES = 4
SUBCORES = 32
SC_ROWS = 64

_H1_OFF = 0
_H2_OFF = 2048
_H3_OFF = 3072
_HTOT = 4096


def _find_crossing(hists, off, nbuckets, m):
    lanes = lax.broadcasted_iota(jnp.int32, (16,), 0)
    zeros16 = jnp.zeros((16,), jnp.int32)

    def body(c, carry):
        found, b, pbef, acc = carry
        vs = [h[pl.ds(off + c * 16, 16)] for h in hists]
        v = vs[0]
        for u in range(1, len(hists)):
            v = v + vs[u]
        for h in hists:
            h[pl.ds(off + c * 16, 16)] = zeros16
        cs = plsc.cumsum(v) + acc
        pb = cs - v
        cross = cs > m
        lane = jnp.min(jnp.where(cross, lanes, BIG))
        this_found = (lane < 16).astype(jnp.int32)
        use = this_found * (1 - found)
        b = jnp.where(use == 1, c * 16 + lane, b)
        pbef = jnp.where(use == 1, jnp.min(jnp.where(cross, pb, BIG)), pbef)
        found = jnp.maximum(found, this_found)
        acc = acc + jnp.sum(v)
        return found, b, pbef, acc

    init = (jnp.int32(0), jnp.int32(0), jnp.int32(0), jnp.int32(0))
    _, b, pbef, _ = lax.fori_loop(0, nbuckets // 16, body, init)
    return b, pbef


def _make_sc_kernel(sc_rows):
    rows_per = sc_rows // SUBCORES
    mesh = plsc.VectorSubcoreMesh(core_axis_name="c", subcore_axis_name="s")

    @lambda body: pl.kernel(
        body,
        out_type=jax.ShapeDtypeStruct((sc_rows, N), jnp.float32),
        mesh=mesh,
        scratch_types=[
            pltpu.VMEM((N,), jnp.float32),
            pltpu.VMEM((N,), jnp.float32),
            pltpu.VMEM((N + 16,), jnp.int32),
        ]
        + [pltpu.VMEM((_HTOT,), jnp.int32) for _ in range(HCOPIES)],
        compiler_params=pltpu.CompilerParams(needs_layout_passes=False),
    )
    def sc_kernel(x_hbm, out_hbm, row_v, out_v, cand, *hists):
        wid = lax.axis_index("s") * 2 + lax.axis_index("c")
        ones = jnp.ones((16,), jnp.int32)
        zeros16 = jnp.zeros((16,), jnp.int32)
        lanes = lax.broadcasted_iota(jnp.int32, (16,), 0)

        def clr(c, _):
            for h in hists:
                h[pl.ds(c * 16, 16)] = zeros16
            return 0

        lax.fori_loop(0, _HTOT // 16, clr, 0)

        def per_row(j, _):
            row = wid * rows_per + j
            pltpu.sync_copy(x_hbm.at[row], row_v)

            def s1(c, _):
                for u in range(UNROLL):
                    v = row_v[pl.ds(c * (16 * UNROLL) + u * 16, 16)]
                    bits = plsc.bitcast(v, jnp.int32) & MASK31
                    b = lax.shift_right_logical(bits, 20)
                    plsc.addupdate_scatter(hists[u % HCOPIES], [b], ones)
                return 0

            lax.fori_loop(0, CHUNKS // UNROLL, s1, 0)

            m1 = jnp.int32(N - TOPK_K)
            b1, pbef1 = _find_crossing(hists, _H1_OFF, 2048, m1)
            m2 = m1 - pbef1

            def s2(c, cnt):
                for u in range(UNROLL):
                    v = row_v[pl.ds(c * (16 * UNROLL) + u * 16, 16)]
                    bits = plsc.bitcast(v, jnp.int32) & MASK31
                    match = lax.shift_right_logical(bits, 20) == b1
                    b = (lax.shift_right_logical(bits, 10) & 1023) + _H2_OFF
                    plsc.addupdate_scatter(hists[u % HCOPIES], [b], ones, mask=match)
                    plsc.store_compressed(cand.at[pl.ds(cnt, 16)], bits, mask=match)
                    cnt = cnt + jnp.sum(match.astype(jnp.int32))
                return cnt

            cnt = lax.fori_loop(0, CHUNKS // UNROLL, s2, jnp.int32(0))
            nc = (cnt + 15) // 16
            b2, pbef2 = _find_crossing(hists, _H2_OFF, 1024, m2)
            m3 = m2 - pbef2
            pfx21 = (b1 << 10) | b2

            def s3(c, _):
                bits = cand[pl.ds(c * 16, 16)]
                valid = (c * 16 + lanes) < cnt
                match = valid & (lax.shift_right_logical(bits, 10) == pfx21)
                b = (bits & 1023) + _H3_OFF
                plsc.addupdate_scatter(hists[0], [b], ones, mask=match)
                return 0

            lax.fori_loop(0, nc, s3, 0)
            b3, _ = _find_crossing(hists, _H3_OFF, 1024, m3)
            thr = (pfx21 << 10) | b3

            def mk(c, _):
                for u in range(UNROLL):
                    v = row_v[pl.ds(c * (16 * UNROLL) + u * 16, 16)]
                    bits = plsc.bitcast(v, jnp.int32) & MASK31
                    out_v[pl.ds(c * (16 * UNROLL) + u * 16, 16)] = jnp.where(
                        bits >= thr, v, 0.0
                    )
                return 0

            lax.fori_loop(0, CHUNKS // UNROLL, mk, 0)
            pltpu.sync_copy(out_v, out_hbm.at[row])
            return 0

        lax.fori_loop(0, rows_per, per_row, 0)

    return sc_kernel


_SC_KERNEL = _make_sc_kernel(SC_ROWS)


def _tc_body(x_ref, o_ref, bits_ref):
    xv = x_ref[...]
    br = xv.shape[0]
    bits_ref[...] = lax.bitcast_convert_type(xv, jnp.int32) & jnp.int32(MASK31)
    k = jnp.int32(TOPK_K)

    def _cnt(bv, cand):
        return jnp.sum((bv >= cand).astype(jnp.int32), axis=1, keepdims=True)

    def pair_step(i, t):
        hi = lax.shift_left(jnp.int32(1), 30 - 2 * i)
        lo = lax.shift_right_logical(hi, 1)
        bv = bits_ref[...]
        c1 = t | hi
        c2 = c1 | lo
        c3 = t | lo
        ok1 = _cnt(bv, c1) >= k
        ok2 = _cnt(bv, c2) >= k
        ok3 = _cnt(bv, c3) >= k
        return jnp.where(ok1, jnp.where(ok2, c2, c1), jnp.where(ok3, c3, t))

    t = lax.fori_loop(0, 15, pair_step, jnp.zeros((br, 1), jnp.int32))
    bv = bits_ref[...]
    c0 = t | jnp.int32(1)
    t = jnp.where(_cnt(bv, c0) >= k, c0, t)
    o_ref[...] = jnp.where(bv >= t, xv, 0.0)


def _tc_call(x):
    rows, n = x.shape
    br = 8
    return pl.pallas_call(
        _tc_body,
        grid=(rows // br,),
        in_specs=[pl.BlockSpec((br, n), lambda i: (i, 0))],
        out_specs=pl.BlockSpec((br, n), lambda i: (i, 0)),
        out_shape=jax.ShapeDtypeStruct(x.shape, x.dtype),
        scratch_shapes=[pltpu.VMEM((br, n), jnp.int32)],
    )(x)


def kernel(x):
    out_sc = _SC_KERNEL(x[ROWS - SC_ROWS:])
    out_tc = _tc_call(x[: ROWS - SC_ROWS])
    return jnp.concatenate([out_tc, out_sc], axis=0)


# speedup vs baseline: 1.7843x; 1.0627x over previous
import jax
import jax.numpy as jnp
from jax import lax
from jax.experimental import pallas as pl
from jax.experimental.pallas import tpu as pltpu
from jax.experimental.pallas import tpu_sc as plsc

TOPK_K = 256
ROWS = 128
N = 32768
CHUNKS = N // 16
MASK31 = 0x7FFFFFFF
BIG = 0x7FFFFFFF
UNROLL = 8
HCOPIES = 4
SUBCORES = 32
SC_ROWS = 64

_H1_OFF = 0
_H2_OFF = 2048
_H3_OFF = 3072
_HTOT = 4096


def _find_crossing(hists, off, nbuckets, m):
    lanes = lax.broadcasted_iota(jnp.int32, (16,), 0)
    zeros16 = jnp.zeros((16,), jnp.int32)

    def body(c, carry):
        found, b, pbef, acc = carry
        vs = [h[pl.ds(off + c * 16, 16)] for h in hists]
        v = vs[0]
        for u in range(1, len(hists)):
            v = v + vs[u]
        for h in hists:
            h[pl.ds(off + c * 16, 16)] = zeros16
        cs = plsc.cumsum(v) + acc
        pb = cs - v
        cross = cs > m
        lane = jnp.min(jnp.where(cross, lanes, BIG))
        this_found = (lane < 16).astype(jnp.int32)
        use = this_found * (1 - found)
        b = jnp.where(use == 1, c * 16 + lane, b)
        pbef = jnp.where(use == 1, jnp.min(jnp.where(cross, pb, BIG)), pbef)
        found = jnp.maximum(found, this_found)
        acc = acc + jnp.sum(v)
        return found, b, pbef, acc

    init = (jnp.int32(0), jnp.int32(0), jnp.int32(0), jnp.int32(0))
    _, b, pbef, _ = lax.fori_loop(0, nbuckets // 16, body, init)
    return b, pbef


def _make_sc_kernel(sc_rows):
    rows_per = sc_rows // SUBCORES
    mesh = plsc.VectorSubcoreMesh(core_axis_name="c", subcore_axis_name="s")

    @lambda body: pl.kernel(
        body,
        out_type=jax.ShapeDtypeStruct((sc_rows, N), jnp.float32),
        mesh=mesh,
        scratch_types=[
            pltpu.VMEM((N,), jnp.float32),
            pltpu.VMEM((N,), jnp.float32),
            pltpu.VMEM((N + 16,), jnp.int32),
        ]
        + [pltpu.VMEM((_HTOT,), jnp.int32) for _ in range(HCOPIES)],
        compiler_params=pltpu.CompilerParams(needs_layout_passes=False),
    )
    def sc_kernel(x_hbm, out_hbm, row_v, out_v, cand, *hists):
        base = ROWS - sc_rows
        wid = lax.axis_index("s") * 2 + lax.axis_index("c")
        ones = jnp.ones((16,), jnp.int32)
        zeros16 = jnp.zeros((16,), jnp.int32)
        lanes = lax.broadcasted_iota(jnp.int32, (16,), 0)

        def clr(c, _):
            for h in hists:
                h[pl.ds(c * 16, 16)] = zeros16
            return 0

        lax.fori_loop(0, _HTOT // 16, clr, 0)

        def per_row(j, _):
            row = wid * rows_per + j
            pltpu.sync_copy(x_hbm.at[base + row], row_v)

            def s1(c, _):
                for u in range(UNROLL):
                    v = row_v[pl.ds(c * (16 * UNROLL) + u * 16, 16)]
                    bits = plsc.bitcast(v, jnp.int32) & MASK31
                    b = lax.shift_right_logical(bits, 20)
                    plsc.addupdate_scatter(hists[u % HCOPIES], [b], ones)
                return 0

            lax.fori_loop(0, CHUNKS // UNROLL, s1, 0)

            m1 = jnp.int32(N - TOPK_K)
            b1, pbef1 = _find_crossing(hists, _H1_OFF, 2048, m1)
            m2 = m1 - pbef1

            def s2(c, cnt):
                for u in range(UNROLL):
                    v = row_v[pl.ds(c * (16 * UNROLL) + u * 16, 16)]
                    bits = plsc.bitcast(v, jnp.int32) & MASK31
                    match = lax.shift_right_logical(bits, 20) == b1
                    b = (lax.shift_right_logical(bits, 10) & 1023) + _H2_OFF
                    plsc.addupdate_scatter(hists[u % HCOPIES], [b], ones, mask=match)
                    plsc.store_compressed(cand.at[pl.ds(cnt, 16)], bits, mask=match)
                    cnt = cnt + jnp.sum(match.astype(jnp.int32))
                return cnt

            cnt = lax.fori_loop(0, CHUNKS // UNROLL, s2, jnp.int32(0))
            nc = (cnt + 15) // 16
            b2, pbef2 = _find_crossing(hists, _H2_OFF, 1024, m2)
            m3 = m2 - pbef2
            pfx21 = (b1 << 10) | b2

            def s3(c, _):
                bits = cand[pl.ds(c * 16, 16)]
                valid = (c * 16 + lanes) < cnt
                match = valid & (lax.shift_right_logical(bits, 10) == pfx21)
                b = (bits & 1023) + _H3_OFF
                plsc.addupdate_scatter(hists[0], [b], ones, mask=match)
                return 0

            lax.fori_loop(0, nc, s3, 0)
            b3, _ = _find_crossing(hists, _H3_OFF, 1024, m3)
            thr = (pfx21 << 10) | b3

            def mk(c, _):
                for u in range(UNROLL):
                    v = row_v[pl.ds(c * (16 * UNROLL) + u * 16, 16)]
                    bits = plsc.bitcast(v, jnp.int32) & MASK31
                    out_v[pl.ds(c * (16 * UNROLL) + u * 16, 16)] = jnp.where(
                        bits >= thr, v, 0.0
                    )
                return 0

            lax.fori_loop(0, CHUNKS // UNROLL, mk, 0)
            pltpu.sync_copy(out_v, out_hbm.at[row])
            return 0

        lax.fori_loop(0, rows_per, per_row, 0)

    return sc_kernel


_SC_KERNEL = _make_sc_kernel(SC_ROWS)


def _tc_body(x_ref, o_ref, bits_ref):
    xv = x_ref[...]
    br = xv.shape[0]
    bits_ref[...] = lax.bitcast_convert_type(xv, jnp.int32) & jnp.int32(MASK31)
    k = jnp.int32(TOPK_K)

    def _cnt(bv, cand):
        return jnp.sum((bv >= cand).astype(jnp.int32), axis=1, keepdims=True)

    def pair_step(i, t):
        hi = lax.shift_left(jnp.int32(1), 30 - 2 * i)
        lo = lax.shift_right_logical(hi, 1)
        bv = bits_ref[...]
        c1 = t | hi
        c2 = c1 | lo
        c3 = t | lo
        ok1 = _cnt(bv, c1) >= k
        ok2 = _cnt(bv, c2) >= k
        ok3 = _cnt(bv, c3) >= k
        return jnp.where(ok1, jnp.where(ok2, c2, c1), jnp.where(ok3, c3, t))

    t = lax.fori_loop(0, 15, pair_step, jnp.zeros((br, 1), jnp.int32))
    bv = bits_ref[...]
    c0 = t | jnp.int32(1)
    t = jnp.where(_cnt(bv, c0) >= k, c0, t)
    o_ref[...] = jnp.where(bv >= t, xv, 0.0)


def _tc_call(x, tc_rows):
    n = x.shape[1]
    br = 8
    return pl.pallas_call(
        _tc_body,
        grid=(tc_rows // br,),
        in_specs=[pl.BlockSpec((br, n), lambda i: (i, 0))],
        out_specs=pl.BlockSpec((br, n), lambda i: (i, 0)),
        out_shape=jax.ShapeDtypeStruct((tc_rows, n), x.dtype),
        scratch_shapes=[pltpu.VMEM((br, n), jnp.int32)],
    )(x)


def kernel(x):
    out_sc = _SC_KERNEL(x)
    out_tc = _tc_call(x, ROWS - SC_ROWS)
    return jnp.concatenate([out_tc, out_sc], axis=0)


# speedup vs baseline: 1.7958x; 1.0064x over previous
import jax
import jax.numpy as jnp
from jax import lax
from jax.experimental import pallas as pl
from jax.experimental.pallas import tpu as pltpu
from jax.experimental.pallas import tpu_sc as plsc

TOPK_K = 256
ROWS = 128
N = 32768
CHUNKS = N // 16
MASK31 = 0x7FFFFFFF
BIG = 0x7FFFFFFF
UNROLL = 8
HCOPIES = 4
SUBCORES = 32
SC_ROWS = 64

_H1_OFF = 0
_H2_OFF = 2048
_H3_OFF = 3072
_HTOT = 4096


def _find_crossing(hists, off, nbuckets, m):
    lanes = lax.broadcasted_iota(jnp.int32, (16,), 0)
    zeros16 = jnp.zeros((16,), jnp.int32)

    def body(c, carry):
        found, b, pbef, acc = carry
        vs = [h[pl.ds(off + c * 16, 16)] for h in hists]
        v = vs[0]
        for u in range(1, len(hists)):
            v = v + vs[u]
        for h in hists:
            h[pl.ds(off + c * 16, 16)] = zeros16
        cs = plsc.cumsum(v) + acc
        pb = cs - v
        cross = cs > m
        lane = jnp.min(jnp.where(cross, lanes, BIG))
        this_found = (lane < 16).astype(jnp.int32)
        use = this_found * (1 - found)
        b = jnp.where(use == 1, c * 16 + lane, b)
        pbef = jnp.where(use == 1, jnp.min(jnp.where(cross, pb, BIG)), pbef)
        found = jnp.maximum(found, this_found)
        acc = acc + jnp.sum(v)
        return found, b, pbef, acc

    init = (jnp.int32(0), jnp.int32(0), jnp.int32(0), jnp.int32(0))
    _, b, pbef, _ = lax.fori_loop(0, nbuckets // 16, body, init)
    return b, pbef


def _make_sc_kernel(sc_rows):
    rows_per = sc_rows // SUBCORES
    mesh = plsc.VectorSubcoreMesh(core_axis_name="c", subcore_axis_name="s")

    @lambda body: pl.kernel(
        body,
        out_type=jax.ShapeDtypeStruct((sc_rows, N), jnp.float32),
        mesh=mesh,
        scratch_types=[
            pltpu.VMEM((N,), jnp.float32),
            pltpu.VMEM((N + 16,), jnp.float32),
            pltpu.VMEM((N + 16,), jnp.int32),
        ]
        + [pltpu.VMEM((_HTOT,), jnp.int32) for _ in range(HCOPIES)],
        compiler_params=pltpu.CompilerParams(needs_layout_passes=False),
    )
    def sc_kernel(x_hbm, out_hbm, row_v, cand_v, cand_i, *hists):
        base = ROWS - sc_rows
        wid = lax.axis_index("s") * 2 + lax.axis_index("c")
        ones = jnp.ones((16,), jnp.int32)
        zeros16 = jnp.zeros((16,), jnp.int32)
        lanes = lax.broadcasted_iota(jnp.int32, (16,), 0)

        def clr(c, _):
            for h in hists:
                h[pl.ds(c * 16, 16)] = zeros16
            return 0

        lax.fori_loop(0, _HTOT // 16, clr, 0)

        def per_row(j, _):
            row = wid * rows_per + j
            pltpu.sync_copy(x_hbm.at[base + row], row_v)

            def s1(c, _):
                for u in range(UNROLL):
                    v = row_v[pl.ds(c * (16 * UNROLL) + u * 16, 16)]
                    bits = plsc.bitcast(v, jnp.int32) & MASK31
                    b = lax.shift_right_logical(bits, 20)
                    plsc.addupdate_scatter(hists[u % HCOPIES], [b], ones)
                return 0

            lax.fori_loop(0, CHUNKS // UNROLL, s1, 0)

            m1 = jnp.int32(N - TOPK_K)
            b1, pbef1 = _find_crossing(hists, _H1_OFF, 2048, m1)
            m2 = m1 - pbef1

            def s2(c, cnt):
                for u in range(UNROLL):
                    off = c * (16 * UNROLL) + u * 16
                    v = row_v[pl.ds(off, 16)]
                    bits = plsc.bitcast(v, jnp.int32) & MASK31
                    top11 = lax.shift_right_logical(bits, 20)
                    match = top11 == b1
                    row_v[pl.ds(off, 16)] = jnp.where(top11 > b1, v, 0.0)
                    b = (lax.shift_right_logical(bits, 10) & 1023) + _H2_OFF
                    plsc.addupdate_scatter(hists[u % HCOPIES], [b], ones, mask=match)
                    plsc.store_compressed(cand_v.at[pl.ds(cnt, 16)], v, mask=match)
                    plsc.store_compressed(
                        cand_i.at[pl.ds(cnt, 16)], lanes + off, mask=match
                    )
                    cnt = cnt + jnp.sum(match.astype(jnp.int32))
                return cnt

            cnt = lax.fori_loop(0, CHUNKS // UNROLL, s2, jnp.int32(0))
            nc = (cnt + 15) // 16
            b2, pbef2 = _find_crossing(hists, _H2_OFF, 1024, m2)
            m3 = m2 - pbef2
            pfx21 = (b1 << 10) | b2

            def s3(c, _):
                bits = plsc.bitcast(cand_v[pl.ds(c * 16, 16)], jnp.int32) & MASK31
                valid = (c * 16 + lanes) < cnt
                match = valid & (lax.shift_right_logical(bits, 10) == pfx21)
                b = (bits & 1023) + _H3_OFF
                plsc.addupdate_scatter(hists[0], [b], ones, mask=match)
                return 0

            lax.fori_loop(0, nc, s3, 0)
            b3, _ = _find_crossing(hists, _H3_OFF, 1024, m3)
            thr = (pfx21 << 10) | b3

            def fx(c, _):
                vals = cand_v[pl.ds(c * 16, 16)]
                bits = plsc.bitcast(vals, jnp.int32) & MASK31
                idx = cand_i[pl.ds(c * 16, 16)]
                keep = ((c * 16 + lanes) < cnt) & (bits >= thr)
                plsc.store_scatter(row_v, [idx], vals, mask=keep)
                return 0

            lax.fori_loop(0, nc, fx, 0)
            pltpu.sync_copy(row_v, out_hbm.at[row])
            return 0

        lax.fori_loop(0, rows_per, per_row, 0)

    return sc_kernel


_SC_KERNEL = _make_sc_kernel(SC_ROWS)


def _tc_body(x_ref, o_ref, bits_ref):
    xv = x_ref[...]
    br = xv.shape[0]
    bits_ref[...] = lax.bitcast_convert_type(xv, jnp.int32) & jnp.int32(MASK31)
    k = jnp.int32(TOPK_K)

    def _cnt(bv, cand):
        return jnp.sum((bv >= cand).astype(jnp.int32), axis=1, keepdims=True)

    def pair_step(i, t):
        hi = lax.shift_left(jnp.int32(1), 30 - 2 * i)
        lo = lax.shift_right_logical(hi, 1)
        bv = bits_ref[...]
        c1 = t | hi
        c2 = c1 | lo
        c3 = t | lo
        ok1 = _cnt(bv, c1) >= k
        ok2 = _cnt(bv, c2) >= k
        ok3 = _cnt(bv, c3) >= k
        return jnp.where(ok1, jnp.where(ok2, c2, c1), jnp.where(ok3, c3, t))

    t = lax.fori_loop(0, 15, pair_step, jnp.zeros((br, 1), jnp.int32))
    bv = bits_ref[...]
    c0 = t | jnp.int32(1)
    t = jnp.where(_cnt(bv, c0) >= k, c0, t)
    o_ref[...] = jnp.where(bv >= t, xv, 0.0)


def _tc_call(x, tc_rows):
    n = x.shape[1]
    br = 8
    return pl.pallas_call(
        _tc_body,
        grid=(tc_rows // br,),
        in_specs=[pl.BlockSpec((br, n), lambda i: (i, 0))],
        out_specs=pl.BlockSpec((br, n), lambda i: (i, 0)),
        out_shape=jax.ShapeDtypeStruct((tc_rows, n), x.dtype),
        scratch_shapes=[pltpu.VMEM((br, n), jnp.int32)],
    )(x)


def kernel(x):
    out_sc = _SC_KERNEL(x)
    out_tc = _tc_call(x, ROWS - SC_ROWS)
    return jnp.concatenate([out_tc, out_sc], axis=0)


# speedup vs baseline: 1.8251x; 1.0163x over previous
import jax
import jax.numpy as jnp
from jax import lax
from jax.experimental import pallas as pl
from jax.experimental.pallas import tpu as pltpu
from jax.experimental.pallas import tpu_sc as plsc

TOPK_K = 256
ROWS = 128
N = 32768
CHUNKS = N // 16
MASK31 = 0x7FFFFFFF
BIG = 0x7FFFFFFF
UNROLL = 8
HCOPIES = 4
SUBCORES = 32
SC_ROWS = 64

_H1_OFF = 0
_H2_OFF = 2048
_H3_OFF = 3072
_HTOT = 4096


def _find_crossing(hists, off, nbuckets, m):
    lanes = lax.broadcasted_iota(jnp.int32, (16,), 0)
    zeros16 = jnp.zeros((16,), jnp.int32)

    def body(c, carry):
        found, b, pbef, acc = carry
        vs = [h[pl.ds(off + c * 16, 16)] for h in hists]
        v = vs[0]
        for u in range(1, len(hists)):
            v = v + vs[u]
        for h in hists:
            h[pl.ds(off + c * 16, 16)] = zeros16
        cs = plsc.cumsum(v) + acc
        pb = cs - v
        cross = cs > m
        lane = jnp.min(jnp.where(cross, lanes, BIG))
        this_found = (lane < 16).astype(jnp.int32)
        use = this_found * (1 - found)
        b = jnp.where(use == 1, c * 16 + lane, b)
        pbef = jnp.where(use == 1, jnp.min(jnp.where(cross, pb, BIG)), pbef)
        found = jnp.maximum(found, this_found)
        acc = acc + jnp.sum(v)
        return found, b, pbef, acc

    init = (jnp.int32(0), jnp.int32(0), jnp.int32(0), jnp.int32(0))
    _, b, pbef, _ = lax.fori_loop(0, nbuckets // 16, body, init)
    return b, pbef


CANDMAX = 8192


def _make_sc_kernel(sc_rows):
    rows_per = sc_rows // SUBCORES
    mesh = plsc.VectorSubcoreMesh(core_axis_name="c", subcore_axis_name="s")

    @lambda body: pl.kernel(
        body,
        out_type=jax.ShapeDtypeStruct((sc_rows, N), jnp.float32),
        mesh=mesh,
        scratch_types=[
            pltpu.VMEM((N,), jnp.float32),
            pltpu.VMEM((N,), jnp.float32),
            pltpu.VMEM((CANDMAX + 16,), jnp.float32),
            pltpu.VMEM((CANDMAX + 16,), jnp.int32),
        ]
        + [pltpu.VMEM((_HTOT,), jnp.int32) for _ in range(HCOPIES)]
        + [pltpu.SemaphoreType.DMA] * 4,
        compiler_params=pltpu.CompilerParams(needs_layout_passes=False),
    )
    def sc_kernel(x_hbm, out_hbm, row_a, row_b, cand_v, cand_i, *rest):
        hists = rest[:HCOPIES]
        sems = rest[HCOPIES:]
        base = ROWS - sc_rows
        wid = lax.axis_index("s") * 2 + lax.axis_index("c")
        ones = jnp.ones((16,), jnp.int32)
        zeros16 = jnp.zeros((16,), jnp.int32)
        lanes = lax.broadcasted_iota(jnp.int32, (16,), 0)
        bufs = (row_a, row_b)
        isems = (sems[0], sems[1])
        osems = (sems[2], sems[3])

        def clr(c, _):
            for h in hists:
                h[pl.ds(c * 16, 16)] = zeros16
            return 0

        lax.fori_loop(0, _HTOT // 16, clr, 0)

        def process(row_v):
            def s1(c, _):
                for u in range(UNROLL):
                    v = row_v[pl.ds(c * (16 * UNROLL) + u * 16, 16)]
                    bits = plsc.bitcast(v, jnp.int32) & MASK31
                    b = lax.shift_right_logical(bits, 20)
                    plsc.addupdate_scatter(hists[u % HCOPIES], [b], ones)
                return 0

            lax.fori_loop(0, CHUNKS // UNROLL, s1, 0)

            m1 = jnp.int32(N - TOPK_K)
            b1, pbef1 = _find_crossing(hists, _H1_OFF, 2048, m1)
            m2 = m1 - pbef1

            def s2(c, cnt):
                for u in range(UNROLL):
                    off = c * (16 * UNROLL) + u * 16
                    v = row_v[pl.ds(off, 16)]
                    bits = plsc.bitcast(v, jnp.int32) & MASK31
                    top11 = lax.shift_right_logical(bits, 20)
                    match = top11 == b1
                    row_v[pl.ds(off, 16)] = jnp.where(top11 > b1, v, 0.0)
                    b = (lax.shift_right_logical(bits, 10) & 1023) + _H2_OFF
                    plsc.addupdate_scatter(hists[u % HCOPIES], [b], ones, mask=match)
                    plsc.store_compressed(cand_v.at[pl.ds(cnt, 16)], v, mask=match)
                    plsc.store_compressed(
                        cand_i.at[pl.ds(cnt, 16)], lanes + off, mask=match
                    )
                    cnt = cnt + jnp.sum(match.astype(jnp.int32))
                return cnt

            cnt = lax.fori_loop(0, CHUNKS // UNROLL, s2, jnp.int32(0))
            nc = (cnt + 15) // 16
            b2, pbef2 = _find_crossing(hists, _H2_OFF, 1024, m2)
            m3 = m2 - pbef2
            pfx21 = (b1 << 10) | b2

            def s3(c, _):
                bits = plsc.bitcast(cand_v[pl.ds(c * 16, 16)], jnp.int32) & MASK31
                valid = (c * 16 + lanes) < cnt
                match = valid & (lax.shift_right_logical(bits, 10) == pfx21)
                b = (bits & 1023) + _H3_OFF
                plsc.addupdate_scatter(hists[0], [b], ones, mask=match)
                return 0

            lax.fori_loop(0, nc, s3, 0)
            b3, _ = _find_crossing(hists, _H3_OFF, 1024, m3)
            thr = (pfx21 << 10) | b3

            def fx(c, _):
                vals = cand_v[pl.ds(c * 16, 16)]
                bits = plsc.bitcast(vals, jnp.int32) & MASK31
                idx = cand_i[pl.ds(c * 16, 16)]
                keep = ((c * 16 + lanes) < cnt) & (bits >= thr)
                plsc.store_scatter(row_v, [idx], vals, mask=keep)
                return 0

            lax.fori_loop(0, nc, fx, 0)

        r0 = wid * rows_per
        hin = {0: pltpu.async_copy(x_hbm.at[base + r0], bufs[0], isems[0])}
        hout = {}
        for j in range(rows_per):
            row = r0 + j
            buf = bufs[j % 2]
            hin[j].wait()
            if j + 1 < rows_per:
                hin[j + 1] = pltpu.async_copy(
                    x_hbm.at[base + row + 1], bufs[(j + 1) % 2], isems[(j + 1) % 2]
                )
            if j >= 2:
                hout[j - 2].wait()
            process(buf)
            hout[j] = pltpu.async_copy(buf, out_hbm.at[row], osems[j % 2])
        for j in range(max(0, rows_per - 2), rows_per):
            hout[j].wait()

    return sc_kernel


_SC_KERNEL = _make_sc_kernel(SC_ROWS)


def _tc_body(x_ref, o_ref, bits_ref):
    xv = x_ref[...]
    br = xv.shape[0]
    bits_ref[...] = lax.bitcast_convert_type(xv, jnp.int32) & jnp.int32(MASK31)
    k = jnp.int32(TOPK_K)

    def _cnt(bv, cand):
        return jnp.sum((bv >= cand).astype(jnp.int32), axis=1, keepdims=True)

    def pair_step(i, t):
        hi = lax.shift_left(jnp.int32(1), 30 - 2 * i)
        lo = lax.shift_right_logical(hi, 1)
        bv = bits_ref[...]
        c1 = t | hi
        c2 = c1 | lo
        c3 = t | lo
        ok1 = _cnt(bv, c1) >= k
        ok2 = _cnt(bv, c2) >= k
        ok3 = _cnt(bv, c3) >= k
        return jnp.where(ok1, jnp.where(ok2, c2, c1), jnp.where(ok3, c3, t))

    t = lax.fori_loop(0, 15, pair_step, jnp.zeros((br, 1), jnp.int32))
    bv = bits_ref[...]
    c0 = t | jnp.int32(1)
    t = jnp.where(_cnt(bv, c0) >= k, c0, t)
    o_ref[...] = jnp.where(bv >= t, xv, 0.0)


def _tc_call(x, tc_rows):
    n = x.shape[1]
    br = 8
    return pl.pallas_call(
        _tc_body,
        grid=(tc_rows // br,),
        in_specs=[pl.BlockSpec((br, n), lambda i: (i, 0))],
        out_specs=pl.BlockSpec((br, n), lambda i: (i, 0)),
        out_shape=jax.ShapeDtypeStruct((tc_rows, n), x.dtype),
        scratch_shapes=[pltpu.VMEM((br, n), jnp.int32)],
    )(x)


def kernel(x):
    out_sc = _SC_KERNEL(x)
    out_tc = _tc_call(x, ROWS - SC_ROWS)
    return jnp.concatenate([out_tc, out_sc], axis=0)
